# baseline (speedup 1.0000x reference)
import functools

import jax
import jax.numpy as jnp
from jax import lax
from jax.experimental import pallas as pl
from jax.experimental.pallas import tpu as pltpu
from jax.experimental.pallas import tpu_sc as plsc

N_NODES = 10000
E_EDGES = 320000
CH = 128
NW = 32
EPW = E_EDGES // NW
C = 40
NCHUNK = EPW // C
NPAD = 10240
ROWS_PER_SUB = NPAD // 16


def _proj_body(q_ref, wl_ref, wr_ref, gl_ref, gr_ref):
    x = q_ref[...]
    dn = (((1,), (1,)), ((), ()))
    gl_ref[...] = lax.dot_general(x, wl_ref[...], dn, preferred_element_type=jnp.float32)
    gr_ref[...] = lax.dot_general(x, wr_ref[...], dn, preferred_element_type=jnp.float32)


def _project(q, W_l, W_r):
    grid = 25
    rb = N_NODES // grid
    return pl.pallas_call(
        _proj_body,
        grid=(grid,),
        in_specs=[
            pl.BlockSpec((rb, CH), lambda i: (i, 0)),
            pl.BlockSpec((CH, CH), lambda i: (0, 0)),
            pl.BlockSpec((CH, CH), lambda i: (0, 0)),
        ],
        out_specs=[
            pl.BlockSpec((rb, CH), lambda i: (i, 0)),
            pl.BlockSpec((rb, CH), lambda i: (i, 0)),
        ],
        out_shape=[
            jax.ShapeDtypeStruct((N_NODES, CH), jnp.float32),
            jax.ShapeDtypeStruct((N_NODES, CH), jnp.float32),
        ],
    )(q, W_l, W_r)


def _takev(a, idx):
    dn = lax.GatherDimensionNumbers(offset_dims=(), collapsed_slice_dims=(0,),
                                    start_index_map=(0,))
    return lax.gather(a, idx.reshape(16, 1), dn, (1,),
                      mode=lax.GatherScatterMode.PROMISE_IN_BOUNDS)


def _edge_body(gl_hbm, gr_hbm, src_hbm, dst_hbm, env_hbm, a_hbm, zeros_hbm,
               num_hbm, den_hbm,
               src_a, dst_a, env_a, src_b, dst_b, env_b, dstv,
               glv_a, grv_a, glv_b, grv_b, a_v, denbuf,
               acc_sp, semA1, semA2, semB1, semB2, semIA, semIB):
    cid = lax.axis_index("c")
    sid = lax.axis_index("s")
    wid = sid * 2 + cid
    base = wid * EPW

    pltpu.sync_copy(zeros_hbm.at[pl.ds(sid * ROWS_PER_SUB, ROWS_PER_SUB)],
                    acc_sp.at[pl.ds(sid * ROWS_PER_SUB, ROWS_PER_SUB)])
    pltpu.sync_copy(a_hbm, a_v)

    zero16 = jnp.zeros((16,), jnp.float32)

    def zero_den(i, carry_z):
        denbuf[pl.ds(i * 16, 16)] = zero16
        return carry_z
    lax.fori_loop(0, (NPAD + 16) // 16, zero_den, 0)
    plsc.subcore_barrier()

    a_chunks = [a_v[pl.ds(16 * j, 16)] for j in range(8)]
    iota16 = lax.iota(jnp.int32, 16)
    onehot0 = jnp.where(iota16 == 0, 1.0, 0.0)
    bfly = [lax.bitwise_xor(iota16, k) for k in (8, 4, 2, 1)]

    def issue_idx(off, srcv, dstv_c, envv, sem):
        pltpu.async_copy(src_hbm.at[pl.ds(base + off, C)], srcv, sem)
        pltpu.async_copy(dst_hbm.at[pl.ds(base + off, C)], dstv_c.at[pl.ds(0, C)], sem)
        pltpu.async_copy(env_hbm.at[pl.ds(base + off, C)], envv.at[pl.ds(0, C)], sem)

    def wait_idx(off, srcv, dstv_c, envv, sem):
        pltpu.make_async_copy(src_hbm.at[pl.ds(base + off, C)], srcv, sem).wait()
        pltpu.make_async_copy(dst_hbm.at[pl.ds(base + off, C)], dstv_c.at[pl.ds(0, C)], sem).wait()
        pltpu.make_async_copy(env_hbm.at[pl.ds(base + off, C)], envv.at[pl.ds(0, C)], sem).wait()

    def issue_g(srcv, dstv_c, glv, grv, s1, s2):
        pltpu.async_copy(gl_hbm.at[srcv], glv, s1)
        pltpu.async_copy(gr_hbm.at[dstv_c.at[pl.ds(0, C)]], grv, s2)

    def wait_g(srcv, dstv_c, glv, grv, s1, s2):
        pltpu.make_async_copy(gl_hbm.at[srcv], glv, s1).wait()
        pltpu.make_async_copy(gr_hbm.at[dstv_c.at[pl.ds(0, C)]], grv, s2).wait()

    def compute_and_scatter(glv, grv, dst_c, env_c):
        NE = 4

        def edge_group(i, carry_e):
            es = [NE * i + k for k in range(NE)]
            ts = [[] for _ in es]
            for j in range(8):
                for k, e in enumerate(es):
                    s0 = glv[e, pl.ds(16 * j, 16)] + grv[e, pl.ds(16 * j, 16)]
                    r0 = s0 / (1.0 + jnp.exp(-s0))
                    ts[k].append(r0 * a_chunks[j])
            accs = []
            for k in range(NE):
                t0 = ts[k]
                accs.append(((t0[0] + t0[1]) + (t0[2] + t0[3]))
                            + ((t0[4] + t0[5]) + (t0[6] + t0[7])))
            for idx in bfly:
                accs = [acc + _takev(acc, idx) for acc in accs]
            ws = []
            for k, e in enumerate(es):
                env_e = env_c[pl.ds(e, 16)][0]
                ws.append((env_e + 1e-7) * jnp.exp(accs[k]))
            for j in range(8):
                for k, e in enumerate(es):
                    glv[e, pl.ds(16 * j, 16)] = ws[k] * glv[e, pl.ds(16 * j, 16)]
            for k, e in enumerate(es):
                d0 = dst_c[pl.ds(e, 16)][0]
                tden0 = denbuf[pl.ds(d0, 16)]
                denbuf[pl.ds(d0, 16)] = tden0 + ws[k] * onehot0
            return carry_e
        lax.fori_loop(0, C // NE, edge_group, 0)
        dstv[pl.ds(0, 16)] = dst_c[pl.ds(0, 16)]
        dstv[pl.ds(16, 16)] = dst_c[pl.ds(16, 16)]
        dstv[pl.ds(C - 16, 16)] = dst_c[pl.ds(C - 16, 16)]
        pltpu.sync_copy(glv, acc_sp.at[dstv], add=True)

    pltpu.sync_copy(src_hbm.at[pl.ds(base, C)], src_a)
    pltpu.sync_copy(dst_hbm.at[pl.ds(base, C)], dst_a.at[pl.ds(0, C)])
    pltpu.sync_copy(env_hbm.at[pl.ds(base, C)], env_a.at[pl.ds(0, C)])
    issue_g(src_a, dst_a, glv_a, grv_a, semA1, semA2)
    issue_idx(C, src_b, dst_b, env_b, semIB)

    npair = NCHUNK // 2

    def pair(i, carry):
        offa = (2 * i) * C
        offb = offa + C
        offn = offa + 2 * C
        wait_idx(offb, src_b, dst_b, env_b, semIB)
        issue_g(src_b, dst_b, glv_b, grv_b, semB1, semB2)
        wait_g(src_a, dst_a, glv_a, grv_a, semA1, semA2)
        compute_and_scatter(glv_a, grv_a, dst_a, env_a)

        @pl.when(i < npair - 1)
        def _steady():
            issue_idx(offn, src_a, dst_a, env_a, semIA)

        wait_g(src_b, dst_b, glv_b, grv_b, semB1, semB2)
        compute_and_scatter(glv_b, grv_b, dst_b, env_b)

        @pl.when(i < npair - 1)
        def _steady2():
            wait_idx(offn, src_a, dst_a, env_a, semIA)
            issue_g(src_a, dst_a, glv_a, grv_a, semA1, semA2)
            issue_idx(offn + C, src_b, dst_b, env_b, semIB)
        return carry

    lax.fori_loop(0, npair, pair, 0)

    plsc.subcore_barrier()
    pltpu.sync_copy(acc_sp.at[pl.ds(sid * ROWS_PER_SUB, ROWS_PER_SUB)],
                    num_hbm.at[cid, pl.ds(sid * ROWS_PER_SUB, ROWS_PER_SUB)])
    pltpu.sync_copy(denbuf.at[pl.ds(0, NPAD)], den_hbm.at[wid])


@functools.partial(
    pl.kernel,
    mesh=plsc.VectorSubcoreMesh(core_axis_name="c", subcore_axis_name="s"),
    out_type=[
        jax.ShapeDtypeStruct((2, NPAD, CH), jnp.float32),
        jax.ShapeDtypeStruct((NW, NPAD), jnp.float32),
    ],
    scratch_types=[
        pltpu.VMEM((C,), jnp.int32),
        pltpu.VMEM((C + 16,), jnp.int32),
        pltpu.VMEM((C + 16,), jnp.float32),
        pltpu.VMEM((C,), jnp.int32),
        pltpu.VMEM((C + 16,), jnp.int32),
        pltpu.VMEM((C + 16,), jnp.float32),
        pltpu.VMEM((C,), jnp.int32),
        pltpu.VMEM((C, CH), jnp.float32),
        pltpu.VMEM((C, CH), jnp.float32),
        pltpu.VMEM((C, CH), jnp.float32),
        pltpu.VMEM((C, CH), jnp.float32),
        pltpu.VMEM((CH,), jnp.float32),
        pltpu.VMEM((NPAD + 16,), jnp.float32),
        pltpu.VMEM_SHARED((NPAD, CH), jnp.float32),
        pltpu.SemaphoreType.DMA,
        pltpu.SemaphoreType.DMA,
        pltpu.SemaphoreType.DMA,
        pltpu.SemaphoreType.DMA,
        pltpu.SemaphoreType.DMA,
        pltpu.SemaphoreType.DMA,
    ],
)
def _edge_pass(gl_hbm, gr_hbm, src_hbm, dst_hbm, env_hbm, a_hbm, zeros_hbm,
               num_hbm, den_hbm, *rest):
    _edge_body(gl_hbm, gr_hbm, src_hbm, dst_hbm, env_hbm, a_hbm, zeros_hbm,
               num_hbm, den_hbm, *rest)


def _norm_body(num_ref, den_ref, o_ref):
    p = num_ref[...]
    num = p[0] + p[1]
    d = den_ref[...]
    den = jnp.sum(d, axis=0)
    o_ref[...] = jnp.where(den > 0, num / den, 0.0)


def _normalize(num_partials, den_col):
    grid = 25
    rb = N_NODES // grid
    return pl.pallas_call(
        _norm_body,
        grid=(grid,),
        in_specs=[
            pl.BlockSpec((2, rb, CH), lambda i: (0, i, 0)),
            pl.BlockSpec((NW, rb, 1), lambda i: (0, i, 0)),
        ],
        out_specs=pl.BlockSpec((rb, CH), lambda i: (i, 0)),
        out_shape=jax.ShapeDtypeStruct((N_NODES, CH), jnp.float32),
    )(num_partials, den_col)


def kernel(q, k, v, envelope, edge_index, W_l, W_r, a):
    del k, v
    g_l, g_r = _project(q, W_l, W_r)
    src = edge_index[0].astype(jnp.int32)
    dst = edge_index[1].astype(jnp.int32)
    a_flat = a.reshape(CH).astype(jnp.float32)
    zeros = jnp.zeros((NPAD, CH), jnp.float32)
    num_p, den_p = _edge_pass(g_l, g_r, src, dst, envelope, a_flat, zeros)
    den_col = den_p.reshape(NW, NPAD, 1)
    return _normalize(num_p, den_col)


# speedup vs baseline: 1.4355x; 1.4355x over previous
import functools

import jax
import jax.numpy as jnp
from jax import lax
from jax.experimental import pallas as pl
from jax.experimental.pallas import tpu as pltpu
from jax.experimental.pallas import tpu_sc as plsc

N_NODES = 10000
E_EDGES = 320000
CH = 128
NW = 32
EPW = E_EDGES // NW
C = 40
NCHUNK = EPW // C
NPAD = 10240
ROWS_PER_SUB = NPAD // 16


def _proj_body(q_ref, wl_ref, wr_ref, gl_ref, gr_ref):
    x = q_ref[...]
    dn = (((1,), (1,)), ((), ()))
    gl_ref[...] = lax.dot_general(x, wl_ref[...], dn, preferred_element_type=jnp.float32)
    gr_ref[...] = lax.dot_general(x, wr_ref[...], dn, preferred_element_type=jnp.float32)


def _project(q, W_l, W_r):
    grid = 25
    rb = N_NODES // grid
    return pl.pallas_call(
        _proj_body,
        grid=(grid,),
        in_specs=[
            pl.BlockSpec((rb, CH), lambda i: (i, 0)),
            pl.BlockSpec((CH, CH), lambda i: (0, 0)),
            pl.BlockSpec((CH, CH), lambda i: (0, 0)),
        ],
        out_specs=[
            pl.BlockSpec((rb, CH), lambda i: (i, 0)),
            pl.BlockSpec((rb, CH), lambda i: (i, 0)),
        ],
        out_shape=[
            jax.ShapeDtypeStruct((N_NODES, CH), jnp.float32),
            jax.ShapeDtypeStruct((N_NODES, CH), jnp.float32),
        ],
    )(q, W_l, W_r)


def _takev(a, idx):
    dn = lax.GatherDimensionNumbers(offset_dims=(), collapsed_slice_dims=(0,),
                                    start_index_map=(0,))
    return lax.gather(a, idx.reshape(16, 1), dn, (1,),
                      mode=lax.GatherScatterMode.PROMISE_IN_BOUNDS)


def _edge_body(gl_hbm, gr_hbm, src_hbm, dst_hbm, env_hbm, a_hbm, zeros_hbm,
               num_hbm, den_hbm,
               src_a, dst_a, env_a, src_b, dst_b, env_b, dstv,
               glv_a, grv_a, glv_b, grv_b, outbuf, a_v, denbuf,
               acc_sp, semA1, semA2, semB1, semB2, semIA, semIB):
    cid = lax.axis_index("c")
    sid = lax.axis_index("s")
    wid = sid * 2 + cid
    base = wid * EPW

    pltpu.sync_copy(zeros_hbm.at[pl.ds(sid * ROWS_PER_SUB, ROWS_PER_SUB)],
                    acc_sp.at[pl.ds(sid * ROWS_PER_SUB, ROWS_PER_SUB)])
    pltpu.sync_copy(a_hbm, a_v)

    zero16 = jnp.zeros((16,), jnp.float32)

    def zero_den(i, carry_z):
        denbuf[pl.ds(i * 16, 16)] = zero16
        return carry_z
    lax.fori_loop(0, (NPAD + 16) // 16, zero_den, 0)
    plsc.subcore_barrier()

    a_chunks = [a_v[pl.ds(16 * j, 16)] for j in range(8)]
    iota16 = lax.iota(jnp.int32, 16)
    onehot0 = jnp.where(iota16 == 0, 1.0, 0.0)
    bfly = [lax.bitwise_xor(iota16, k) for k in (8, 4, 2, 1)]

    def issue_idx(off, srcv, dstv_c, envv, sem):
        pltpu.async_copy(src_hbm.at[pl.ds(base + off, C)], srcv, sem)
        pltpu.async_copy(dst_hbm.at[pl.ds(base + off, C)], dstv_c.at[pl.ds(0, C)], sem)
        pltpu.async_copy(env_hbm.at[pl.ds(base + off, C)], envv.at[pl.ds(0, C)], sem)

    def wait_idx(off, srcv, dstv_c, envv, sem):
        pltpu.make_async_copy(src_hbm.at[pl.ds(base + off, C)], srcv, sem).wait()
        pltpu.make_async_copy(dst_hbm.at[pl.ds(base + off, C)], dstv_c.at[pl.ds(0, C)], sem).wait()
        pltpu.make_async_copy(env_hbm.at[pl.ds(base + off, C)], envv.at[pl.ds(0, C)], sem).wait()

    def issue_g(srcv, dstv_c, glv, grv, s1, s2):
        pltpu.async_copy(gl_hbm.at[srcv], glv, s1)
        pltpu.async_copy(gr_hbm.at[dstv_c.at[pl.ds(0, C)]], grv, s2)

    def wait_g(srcv, dstv_c, glv, grv, s1, s2):
        pltpu.make_async_copy(gl_hbm.at[srcv], glv, s1).wait()
        pltpu.make_async_copy(gr_hbm.at[dstv_c.at[pl.ds(0, C)]], grv, s2).wait()

    def compute_and_scatter(glv, grv, dst_c, env_c):
        def edge_pair(i, carry_e):
            e0 = 2 * i
            e1 = e0 + 1
            gl0 = [glv[e0, pl.ds(16 * j, 16)] for j in range(8)]
            gl1 = [glv[e1, pl.ds(16 * j, 16)] for j in range(8)]
            t0 = []
            t1 = []
            for j in range(8):
                s0 = gl0[j] + grv[e0, pl.ds(16 * j, 16)]
                s1 = gl1[j] + grv[e1, pl.ds(16 * j, 16)]
                r0 = s0 / (1.0 + jnp.exp(-s0))
                r1 = s1 / (1.0 + jnp.exp(-s1))
                t0.append(r0 * a_chunks[j])
                t1.append(r1 * a_chunks[j])
            acc0 = ((t0[0] + t0[1]) + (t0[2] + t0[3])) + ((t0[4] + t0[5]) + (t0[6] + t0[7]))
            acc1 = ((t1[0] + t1[1]) + (t1[2] + t1[3])) + ((t1[4] + t1[5]) + (t1[6] + t1[7]))
            for idx in bfly:
                acc0 = acc0 + _takev(acc0, idx)
                acc1 = acc1 + _takev(acc1, idx)
            env_e0 = env_c[pl.ds(e0, 16)][0]
            env_e1 = env_c[pl.ds(e1, 16)][0]
            w0 = (env_e0 + 1e-7) * jnp.exp(acc0)
            w1 = (env_e1 + 1e-7) * jnp.exp(acc1)
            for j in range(8):
                outbuf[e0, pl.ds(16 * j, 16)] = w0 * gl0[j]
                outbuf[e1, pl.ds(16 * j, 16)] = w1 * gl1[j]
            d0 = dst_c[pl.ds(e0, 16)][0]
            tden0 = denbuf[pl.ds(d0, 16)]
            denbuf[pl.ds(d0, 16)] = tden0 + w0 * onehot0
            d1 = dst_c[pl.ds(e1, 16)][0]
            tden1 = denbuf[pl.ds(d1, 16)]
            denbuf[pl.ds(d1, 16)] = tden1 + w1 * onehot0
            return carry_e
        lax.fori_loop(0, C // 2, edge_pair, 0)
        dstv[pl.ds(0, 16)] = dst_c[pl.ds(0, 16)]
        dstv[pl.ds(16, 16)] = dst_c[pl.ds(16, 16)]
        dstv[pl.ds(C - 16, 16)] = dst_c[pl.ds(C - 16, 16)]
        pltpu.sync_copy(outbuf, acc_sp.at[dstv], add=True)

    pltpu.sync_copy(src_hbm.at[pl.ds(base, C)], src_a)
    pltpu.sync_copy(dst_hbm.at[pl.ds(base, C)], dst_a.at[pl.ds(0, C)])
    pltpu.sync_copy(env_hbm.at[pl.ds(base, C)], env_a.at[pl.ds(0, C)])
    issue_g(src_a, dst_a, glv_a, grv_a, semA1, semA2)
    issue_idx(C, src_b, dst_b, env_b, semIB)

    npair = NCHUNK // 2

    def pair(i, carry):
        offa = (2 * i) * C
        offb = offa + C
        offn = offa + 2 * C
        wait_idx(offb, src_b, dst_b, env_b, semIB)
        issue_g(src_b, dst_b, glv_b, grv_b, semB1, semB2)
        wait_g(src_a, dst_a, glv_a, grv_a, semA1, semA2)
        compute_and_scatter(glv_a, grv_a, dst_a, env_a)

        @pl.when(i < npair - 1)
        def _steady():
            issue_idx(offn, src_a, dst_a, env_a, semIA)

        wait_g(src_b, dst_b, glv_b, grv_b, semB1, semB2)
        compute_and_scatter(glv_b, grv_b, dst_b, env_b)

        @pl.when(i < npair - 1)
        def _steady2():
            wait_idx(offn, src_a, dst_a, env_a, semIA)
            issue_g(src_a, dst_a, glv_a, grv_a, semA1, semA2)
            issue_idx(offn + C, src_b, dst_b, env_b, semIB)
        return carry

    lax.fori_loop(0, npair, pair, 0)

    plsc.subcore_barrier()
    pltpu.sync_copy(acc_sp.at[pl.ds(sid * ROWS_PER_SUB, ROWS_PER_SUB)],
                    num_hbm.at[cid, pl.ds(sid * ROWS_PER_SUB, ROWS_PER_SUB)])
    pltpu.sync_copy(denbuf.at[pl.ds(0, NPAD)], den_hbm.at[wid])


@functools.partial(
    pl.kernel,
    mesh=plsc.VectorSubcoreMesh(core_axis_name="c", subcore_axis_name="s"),
    out_type=[
        jax.ShapeDtypeStruct((2, NPAD, CH), jnp.float32),
        jax.ShapeDtypeStruct((NW, NPAD), jnp.float32),
    ],
    scratch_types=[
        pltpu.VMEM((C,), jnp.int32),
        pltpu.VMEM((C + 16,), jnp.int32),
        pltpu.VMEM((C + 16,), jnp.float32),
        pltpu.VMEM((C,), jnp.int32),
        pltpu.VMEM((C + 16,), jnp.int32),
        pltpu.VMEM((C + 16,), jnp.float32),
        pltpu.VMEM((C,), jnp.int32),
        pltpu.VMEM((C, CH), jnp.float32),
        pltpu.VMEM((C, CH), jnp.float32),
        pltpu.VMEM((C, CH), jnp.float32),
        pltpu.VMEM((C, CH), jnp.float32),
        pltpu.VMEM((C, CH), jnp.float32),
        pltpu.VMEM((CH,), jnp.float32),
        pltpu.VMEM((NPAD + 16,), jnp.float32),
        pltpu.VMEM_SHARED((NPAD, CH), jnp.float32),
        pltpu.SemaphoreType.DMA,
        pltpu.SemaphoreType.DMA,
        pltpu.SemaphoreType.DMA,
        pltpu.SemaphoreType.DMA,
        pltpu.SemaphoreType.DMA,
        pltpu.SemaphoreType.DMA,
    ],
)
def _edge_pass(gl_hbm, gr_hbm, src_hbm, dst_hbm, env_hbm, a_hbm, zeros_hbm,
               num_hbm, den_hbm, *rest):
    _edge_body(gl_hbm, gr_hbm, src_hbm, dst_hbm, env_hbm, a_hbm, zeros_hbm,
               num_hbm, den_hbm, *rest)


def _norm_body(num_ref, den_ref, o_ref):
    p = num_ref[...]
    num = p[0] + p[1]
    d = den_ref[...]
    den = jnp.sum(d, axis=0)
    o_ref[...] = jnp.where(den > 0, num / den, 0.0)


def _normalize(num_partials, den_col):
    grid = 25
    rb = N_NODES // grid
    return pl.pallas_call(
        _norm_body,
        grid=(grid,),
        in_specs=[
            pl.BlockSpec((2, rb, CH), lambda i: (0, i, 0)),
            pl.BlockSpec((NW, rb, 1), lambda i: (0, i, 0)),
        ],
        out_specs=pl.BlockSpec((rb, CH), lambda i: (i, 0)),
        out_shape=jax.ShapeDtypeStruct((N_NODES, CH), jnp.float32),
    )(num_partials, den_col)


def kernel(q, k, v, envelope, edge_index, W_l, W_r, a):
    del k, v
    g_l, g_r = _project(q, W_l, W_r)
    src = edge_index[0].astype(jnp.int32)
    dst = edge_index[1].astype(jnp.int32)
    a_flat = a.reshape(CH).astype(jnp.float32)
    zeros = jnp.zeros((NPAD, CH), jnp.float32)
    num_p, den_p = _edge_pass(g_l, g_r, src, dst, envelope, a_flat, zeros)
    den_col = den_p.reshape(NW, NPAD, 1)
    return _normalize(num_p, den_col)


# speedup vs baseline: 1.4431x; 1.0053x over previous
import functools

import jax
import jax.numpy as jnp
from jax import lax
from jax.experimental import pallas as pl
from jax.experimental.pallas import tpu as pltpu
from jax.experimental.pallas import tpu_sc as plsc

N_NODES = 10000
E_EDGES = 320000
CH = 128
NW = 32
EPW = E_EDGES // NW
C = 40
NCHUNK = EPW // C
NPAD = 10240
ROWS_PER_SUB = NPAD // 16


def _proj_body(q_ref, wl_ref, wr_ref, gl_ref, gr_ref):
    x = q_ref[...]
    dn = (((1,), (1,)), ((), ()))
    gl_ref[...] = lax.dot_general(x, wl_ref[...], dn, preferred_element_type=jnp.float32)
    gr_ref[...] = lax.dot_general(x, wr_ref[...], dn, preferred_element_type=jnp.float32)


def _project(q, W_l, W_r):
    grid = 25
    rb = N_NODES // grid
    return pl.pallas_call(
        _proj_body,
        grid=(grid,),
        in_specs=[
            pl.BlockSpec((rb, CH), lambda i: (i, 0)),
            pl.BlockSpec((CH, CH), lambda i: (0, 0)),
            pl.BlockSpec((CH, CH), lambda i: (0, 0)),
        ],
        out_specs=[
            pl.BlockSpec((rb, CH), lambda i: (i, 0)),
            pl.BlockSpec((rb, CH), lambda i: (i, 0)),
        ],
        out_shape=[
            jax.ShapeDtypeStruct((N_NODES, CH), jnp.float32),
            jax.ShapeDtypeStruct((N_NODES, CH), jnp.float32),
        ],
    )(q, W_l, W_r)


def _takev(a, idx):
    dn = lax.GatherDimensionNumbers(offset_dims=(), collapsed_slice_dims=(0,),
                                    start_index_map=(0,))
    return lax.gather(a, idx.reshape(16, 1), dn, (1,),
                      mode=lax.GatherScatterMode.PROMISE_IN_BOUNDS)


def _edge_body(gl_hbm, gr_hbm, src_hbm, dst_hbm, env_hbm, a_hbm, zeros_hbm,
               num_hbm, den_hbm,
               src_a, dst_a, env_a, src_b, dst_b, env_b, dstv,
               glv_a, grv_a, glv_b, grv_b, outbuf, a_v, denbuf,
               acc_sp, semA1, semA2, semB1, semB2, semIA, semIB):
    cid = lax.axis_index("c")
    sid = lax.axis_index("s")
    wid = sid * 2 + cid
    base = wid * EPW

    pltpu.sync_copy(zeros_hbm.at[pl.ds(sid * ROWS_PER_SUB, ROWS_PER_SUB)],
                    acc_sp.at[pl.ds(sid * ROWS_PER_SUB, ROWS_PER_SUB)])
    pltpu.sync_copy(a_hbm, a_v)

    zero16 = jnp.zeros((16,), jnp.float32)

    def zero_den(i, carry_z):
        denbuf[pl.ds(i * 16, 16)] = zero16
        return carry_z
    lax.fori_loop(0, (NPAD + 16) // 16, zero_den, 0)
    plsc.subcore_barrier()

    a_chunks = [a_v[pl.ds(16 * j, 16)] for j in range(8)]
    iota16 = lax.iota(jnp.int32, 16)
    onehot0 = jnp.where(iota16 == 0, 1.0, 0.0)
    bfly = [lax.bitwise_xor(iota16, k) for k in (8, 4, 2, 1)]

    def issue_idx(off, srcv, dstv_c, envv, sem):
        pltpu.async_copy(src_hbm.at[pl.ds(base + off, C)], srcv, sem)
        pltpu.async_copy(dst_hbm.at[pl.ds(base + off, C)], dstv_c.at[pl.ds(0, C)], sem)
        pltpu.async_copy(env_hbm.at[pl.ds(base + off, C)], envv.at[pl.ds(0, C)], sem)

    def wait_idx(off, srcv, dstv_c, envv, sem):
        pltpu.make_async_copy(src_hbm.at[pl.ds(base + off, C)], srcv, sem).wait()
        pltpu.make_async_copy(dst_hbm.at[pl.ds(base + off, C)], dstv_c.at[pl.ds(0, C)], sem).wait()
        pltpu.make_async_copy(env_hbm.at[pl.ds(base + off, C)], envv.at[pl.ds(0, C)], sem).wait()

    def issue_g(srcv, dstv_c, glv, grv, s1, s2):
        pltpu.async_copy(gl_hbm.at[srcv], glv, s1)
        pltpu.async_copy(gr_hbm.at[dstv_c.at[pl.ds(0, C)]], grv, s2)

    def wait_g(srcv, dstv_c, glv, grv, s1, s2):
        pltpu.make_async_copy(gl_hbm.at[srcv], glv, s1).wait()
        pltpu.make_async_copy(gr_hbm.at[dstv_c.at[pl.ds(0, C)]], grv, s2).wait()

    def compute_and_scatter(glv, grv, dst_c, env_c):
        def edge_pair(i, carry_e):
            e0 = 2 * i
            e1 = e0 + 1
            gl0 = [glv[e0, pl.ds(16 * j, 16)] for j in range(8)]
            gl1 = [glv[e1, pl.ds(16 * j, 16)] for j in range(8)]
            t0 = []
            t1 = []
            for j in range(8):
                s0 = gl0[j] + grv[e0, pl.ds(16 * j, 16)]
                s1 = gl1[j] + grv[e1, pl.ds(16 * j, 16)]
                es0 = jnp.exp(s0)
                es1 = jnp.exp(s1)
                r0 = (s0 * es0) / (1.0 + es0)
                r1 = (s1 * es1) / (1.0 + es1)
                t0.append(r0 * a_chunks[j])
                t1.append(r1 * a_chunks[j])
            acc0 = ((t0[0] + t0[1]) + (t0[2] + t0[3])) + ((t0[4] + t0[5]) + (t0[6] + t0[7]))
            acc1 = ((t1[0] + t1[1]) + (t1[2] + t1[3])) + ((t1[4] + t1[5]) + (t1[6] + t1[7]))
            for idx in bfly:
                acc0 = acc0 + _takev(acc0, idx)
                acc1 = acc1 + _takev(acc1, idx)
            env_e0 = env_c[pl.ds(e0, 16)][0]
            env_e1 = env_c[pl.ds(e1, 16)][0]
            w0 = (env_e0 + 1e-7) * jnp.exp(acc0)
            w1 = (env_e1 + 1e-7) * jnp.exp(acc1)
            for j in range(8):
                outbuf[e0, pl.ds(16 * j, 16)] = w0 * gl0[j]
                outbuf[e1, pl.ds(16 * j, 16)] = w1 * gl1[j]
            d0 = dst_c[pl.ds(e0, 16)][0]
            tden0 = denbuf[pl.ds(d0, 16)]
            denbuf[pl.ds(d0, 16)] = tden0 + w0 * onehot0
            d1 = dst_c[pl.ds(e1, 16)][0]
            tden1 = denbuf[pl.ds(d1, 16)]
            denbuf[pl.ds(d1, 16)] = tden1 + w1 * onehot0
            return carry_e
        lax.fori_loop(0, C // 2, edge_pair, 0)
        dstv[pl.ds(0, 16)] = dst_c[pl.ds(0, 16)]
        dstv[pl.ds(16, 16)] = dst_c[pl.ds(16, 16)]
        dstv[pl.ds(C - 16, 16)] = dst_c[pl.ds(C - 16, 16)]
        pltpu.sync_copy(outbuf, acc_sp.at[dstv], add=True)

    pltpu.sync_copy(src_hbm.at[pl.ds(base, C)], src_a)
    pltpu.sync_copy(dst_hbm.at[pl.ds(base, C)], dst_a.at[pl.ds(0, C)])
    pltpu.sync_copy(env_hbm.at[pl.ds(base, C)], env_a.at[pl.ds(0, C)])
    issue_g(src_a, dst_a, glv_a, grv_a, semA1, semA2)
    issue_idx(C, src_b, dst_b, env_b, semIB)

    npair = NCHUNK // 2

    def pair(i, carry):
        offa = (2 * i) * C
        offb = offa + C
        offn = offa + 2 * C
        wait_idx(offb, src_b, dst_b, env_b, semIB)
        issue_g(src_b, dst_b, glv_b, grv_b, semB1, semB2)
        wait_g(src_a, dst_a, glv_a, grv_a, semA1, semA2)
        compute_and_scatter(glv_a, grv_a, dst_a, env_a)

        @pl.when(i < npair - 1)
        def _steady():
            issue_idx(offn, src_a, dst_a, env_a, semIA)

        wait_g(src_b, dst_b, glv_b, grv_b, semB1, semB2)
        compute_and_scatter(glv_b, grv_b, dst_b, env_b)

        @pl.when(i < npair - 1)
        def _steady2():
            wait_idx(offn, src_a, dst_a, env_a, semIA)
            issue_g(src_a, dst_a, glv_a, grv_a, semA1, semA2)
            issue_idx(offn + C, src_b, dst_b, env_b, semIB)
        return carry

    lax.fori_loop(0, npair, pair, 0)

    plsc.subcore_barrier()
    pltpu.sync_copy(acc_sp.at[pl.ds(sid * ROWS_PER_SUB, ROWS_PER_SUB)],
                    num_hbm.at[cid, pl.ds(sid * ROWS_PER_SUB, ROWS_PER_SUB)])
    pltpu.sync_copy(denbuf.at[pl.ds(0, NPAD)], den_hbm.at[wid])


@functools.partial(
    pl.kernel,
    mesh=plsc.VectorSubcoreMesh(core_axis_name="c", subcore_axis_name="s"),
    out_type=[
        jax.ShapeDtypeStruct((2, NPAD, CH), jnp.float32),
        jax.ShapeDtypeStruct((NW, NPAD), jnp.float32),
    ],
    scratch_types=[
        pltpu.VMEM((C,), jnp.int32),
        pltpu.VMEM((C + 16,), jnp.int32),
        pltpu.VMEM((C + 16,), jnp.float32),
        pltpu.VMEM((C,), jnp.int32),
        pltpu.VMEM((C + 16,), jnp.int32),
        pltpu.VMEM((C + 16,), jnp.float32),
        pltpu.VMEM((C,), jnp.int32),
        pltpu.VMEM((C, CH), jnp.float32),
        pltpu.VMEM((C, CH), jnp.float32),
        pltpu.VMEM((C, CH), jnp.float32),
        pltpu.VMEM((C, CH), jnp.float32),
        pltpu.VMEM((C, CH), jnp.float32),
        pltpu.VMEM((CH,), jnp.float32),
        pltpu.VMEM((NPAD + 16,), jnp.float32),
        pltpu.VMEM_SHARED((NPAD, CH), jnp.float32),
        pltpu.SemaphoreType.DMA,
        pltpu.SemaphoreType.DMA,
        pltpu.SemaphoreType.DMA,
        pltpu.SemaphoreType.DMA,
        pltpu.SemaphoreType.DMA,
        pltpu.SemaphoreType.DMA,
    ],
)
def _edge_pass(gl_hbm, gr_hbm, src_hbm, dst_hbm, env_hbm, a_hbm, zeros_hbm,
               num_hbm, den_hbm, *rest):
    _edge_body(gl_hbm, gr_hbm, src_hbm, dst_hbm, env_hbm, a_hbm, zeros_hbm,
               num_hbm, den_hbm, *rest)


def _norm_body(num_ref, den_ref, o_ref):
    p = num_ref[...]
    num = p[0] + p[1]
    d = den_ref[...]
    den = jnp.sum(d, axis=0)
    o_ref[...] = jnp.where(den > 0, num / den, 0.0)


def _normalize(num_partials, den_col):
    grid = 25
    rb = N_NODES // grid
    return pl.pallas_call(
        _norm_body,
        grid=(grid,),
        in_specs=[
            pl.BlockSpec((2, rb, CH), lambda i: (0, i, 0)),
            pl.BlockSpec((NW, rb, 1), lambda i: (0, i, 0)),
        ],
        out_specs=pl.BlockSpec((rb, CH), lambda i: (i, 0)),
        out_shape=jax.ShapeDtypeStruct((N_NODES, CH), jnp.float32),
    )(num_partials, den_col)


def kernel(q, k, v, envelope, edge_index, W_l, W_r, a):
    del k, v
    g_l, g_r = _project(q, W_l, W_r)
    src = edge_index[0].astype(jnp.int32)
    dst = edge_index[1].astype(jnp.int32)
    a_flat = a.reshape(CH).astype(jnp.float32)
    zeros = jnp.zeros((NPAD, CH), jnp.float32)
    num_p, den_p = _edge_pass(g_l, g_r, src, dst, envelope, a_flat, zeros)
    den_col = den_p.reshape(NW, NPAD, 1)
    return _normalize(num_p, den_col)
